# Initial kernel scaffold; baseline (speedup 1.0000x reference)
import jax
import jax.numpy as jnp
from jax.experimental import pallas as pl


def kernel(inputs, embed_weight):
    raise NotImplementedError("write your pallas kernel here")


# speedup vs baseline: 3.3297x; 3.3297x over previous
import functools

import jax
import jax.numpy as jnp
from jax import lax
from jax.experimental import pallas as pl
from jax.experimental.pallas import tpu as pltpu
from jax.experimental.pallas import tpu_sc as plsc

EMBED_DIM = 128
CHUNK = 128


@functools.partial(jax.jit, static_argnames=("num_workers", "n_chunks"))
def _sc_embed_lookup(idx, table, *, num_workers, n_chunks):
    mesh = plsc.VectorSubcoreMesh(core_axis_name="c", subcore_axis_name="s")
    num_cores = mesh.num_cores
    total_rows = num_workers * n_chunks * CHUNK
    rows_per_worker = n_chunks * CHUNK

    @functools.partial(
        pl.kernel,
        out_type=jax.ShapeDtypeStruct((total_rows, EMBED_DIM), jnp.float32),
        mesh=mesh,
        scratch_types=[
            pltpu.VMEM((n_chunks, CHUNK), jnp.int32),
            pltpu.VMEM((CHUNK, EMBED_DIM), jnp.float32),
            pltpu.VMEM((CHUNK, EMBED_DIM), jnp.float32),
            pltpu.SemaphoreType.DMA,
            pltpu.SemaphoreType.DMA,
        ],
    )
    def body(idx_hbm, tab_hbm, out_hbm, idx_v, buf0, buf1, sem0, sem1):
        wid = lax.axis_index("s") * num_cores + lax.axis_index("c")
        base = wid * rows_per_worker
        pltpu.sync_copy(idx_hbm.at[wid], idx_v)
        pltpu.async_copy(tab_hbm.at[idx_v.at[0]], buf0, sem0)

        def step(t, carry):
            j0 = 2 * t
            pltpu.async_copy(tab_hbm.at[idx_v.at[j0 + 1]], buf1, sem1)
            pltpu.make_async_copy(tab_hbm.at[idx_v.at[j0]], buf0, sem0).wait()
            pltpu.sync_copy(buf0, out_hbm.at[pl.ds(base + j0 * CHUNK, CHUNK)])

            @pl.when(t + 1 < n_chunks // 2)
            def _():
                pltpu.async_copy(tab_hbm.at[idx_v.at[j0 + 2]], buf0, sem0)

            pltpu.make_async_copy(
                tab_hbm.at[idx_v.at[j0 + 1]], buf1, sem1
            ).wait()
            pltpu.sync_copy(
                buf1, out_hbm.at[pl.ds(base + (j0 + 1) * CHUNK, CHUNK)]
            )
            return carry

        lax.fori_loop(0, n_chunks // 2, step, 0)

    return body(idx, table)


def kernel(inputs, embed_weight):
    if inputs.shape[-1] == 1:
        inputs = jnp.squeeze(inputs, axis=-1)
    lead_shape = inputs.shape
    flat = inputs.reshape(-1).astype(jnp.int32)
    n = flat.shape[0]
    num_workers = 32
    assert n % (num_workers * CHUNK) == 0
    n_chunks = n // (num_workers * CHUNK)
    assert n_chunks % 2 == 0
    idx = flat.reshape(num_workers, n_chunks, CHUNK)
    out = _sc_embed_lookup(
        idx, embed_weight, num_workers=num_workers, n_chunks=n_chunks
    )
    return out.reshape(*lead_shape, EMBED_DIM)
